# baseline (speedup 1.0000x reference)
import functools
import jax
import jax.numpy as jnp
from jax import lax
from jax.experimental import pallas as pl
from jax.experimental.pallas import tpu as pltpu, tpu_sc as plsc

D = 1024
E = 8
H = 4096
N = 2048
A = 2 * N
BT = 128
NB = 40
S = NB * BT
BH = 512
NH = H // BH
NW = 32
APW = A // NW
TPW = N // NW

_sc_mesh = plsc.VectorSubcoreMesh(core_axis_name="c", subcore_axis_name="s")


def _iota16():
    return lax.iota(jnp.int32, 16)


def _take(arr, idx):
    return lax.gather(
        arr, idx[:, None],
        dimension_numbers=lax.GatherDimensionNumbers(
            offset_dims=(), collapsed_slice_dims=(0,), start_index_map=(0,)),
        slice_sizes=(1,),
        mode=lax.GatherScatterMode.PROMISE_IN_BOUNDS)


def _cumsum16(x):
    iota = _iota16()
    s = x
    for sh in (1, 2, 4, 8):
        idx = jnp.maximum(iota - sh, 0)
        s = s + jnp.where(iota >= sh, _take(s, idx), 0)
    return s


def _bcast_at(s, i):
    return _take(s, jnp.full((16,), i, jnp.int32))


def _router_body(x_ref, wr_ref, br_ref, wn_ref, bn_ref, noise_ref,
                 idx_ref, gates_ref):
    xb = x_ref[...]
    logits = jnp.dot(xb, wr_ref[...]) + br_ref[...]
    nlog = jnp.dot(xb, wn_ref[...]) + bn_ref[...]
    sp = jnp.maximum(nlog, 0.0) + jnp.log1p(jnp.exp(-jnp.abs(nlog)))
    nl = logits + noise_ref[...] * sp
    lane = jax.lax.broadcasted_iota(jnp.int32, nl.shape, 1)
    m1 = jnp.max(nl, axis=1, keepdims=True)
    i1 = jnp.min(jnp.where(nl == m1, lane, E), axis=1, keepdims=True)
    nl2 = jnp.where(lane == i1, -jnp.inf, nl)
    m2 = jnp.max(nl2, axis=1, keepdims=True)
    i2 = jnp.min(jnp.where(nl2 == m2, lane, E), axis=1, keepdims=True)
    e2 = jnp.exp(m2 - m1)
    denom = 1.0 + e2
    idx_ref[...] = jnp.concatenate([i1, i2], axis=1)
    gates_ref[...] = jnp.concatenate([1.0 / denom, e2 / denom], axis=1)


def _router(x, W_route, b_route, W_noise, b_noise, noise):
    return pl.pallas_call(
        _router_body,
        out_shape=(jax.ShapeDtypeStruct((N, 2), jnp.int32),
                   jax.ShapeDtypeStruct((N, 2), jnp.float32)),
    )(x, W_route, b_route.reshape(1, E), W_noise, b_noise.reshape(1, E),
      noise)


@functools.partial(
    pl.kernel, mesh=_sc_mesh,
    out_type=(jax.ShapeDtypeStruct((NW, 16), jnp.int32),
              jax.ShapeDtypeStruct((A,), jnp.int32)),
    scratch_types=[
        pltpu.VMEM((APW,), jnp.int32),
        pltpu.VMEM((APW,), jnp.int32),
        pltpu.VMEM((16,), jnp.int32),
        pltpu.SemaphoreType.DMA,
    ],
)
def _sc_counts(idx_hbm, cnt_hbm, rank_hbm, ev, rankv, cntv, sem):
    w = lax.axis_index("s") * 2 + lax.axis_index("c")
    base_a = w * APW
    lanes = _iota16()
    pltpu.sync_copy(idx_hbm.at[pl.ds(base_a, APW)], ev)

    run = jnp.zeros((16,), jnp.int32)
    for c in range(APW // 16):
        v = ev[pl.ds(c * 16, 16)]
        base_r = _take(run, v)
        rank = jnp.zeros((16,), jnp.int32)
        for e in range(E):
            m = v == e
            cs = _cumsum16(jnp.where(m, 1, 0))
            rank = rank + jnp.where(m, cs - 1, 0)
            run = run + jnp.where(lanes == e, _bcast_at(cs, 15), 0)
        rankv[pl.ds(c * 16, 16)] = base_r + rank

    cntv[...] = run
    pltpu.sync_copy(cntv, cnt_hbm.at[w])
    pltpu.sync_copy(rankv, rank_hbm.at[pl.ds(base_a, APW)])


@functools.partial(
    pl.kernel, mesh=_sc_mesh,
    out_type=(jax.ShapeDtypeStruct((S, D), jnp.float32),
              jax.ShapeDtypeStruct((A,), jnp.int32),
              jax.ShapeDtypeStruct((64,), jnp.int32)),
    scratch_types=[
        pltpu.VMEM((APW,), jnp.int32),
        pltpu.VMEM((APW,), jnp.int32),
        pltpu.VMEM((NW, 16), jnp.int32),
        pltpu.VMEM((APW,), jnp.int32),
        pltpu.VMEM((4, 32), jnp.int32),
        pltpu.VMEM((4, 32), jnp.int32),
        pltpu.VMEM((32, D), jnp.float32),
        pltpu.VMEM((64,), jnp.int32),
        pltpu.SemaphoreType.DMA,
    ],
)
def _sc_dispatch(idx_hbm, rank_hbm, cnt_hbm, x_hbm, xg_hbm, slots_hbm,
                 meta_hbm, ev, rankv, allcnt, slotv, tok2d, slot2d, rows,
                 metav, sem):
    w = lax.axis_index("s") * 2 + lax.axis_index("c")
    base_a = w * APW
    pltpu.sync_copy(idx_hbm.at[pl.ds(base_a, APW)], ev)
    pltpu.sync_copy(rank_hbm.at[pl.ds(base_a, APW)], rankv)
    pltpu.sync_copy(cnt_hbm, allcnt)

    base = jnp.zeros((16,), jnp.int32)
    tot = jnp.zeros((16,), jnp.int32)
    for wp in range(NW):
        row = allcnt[wp]
        base = base + jnp.where(wp < w, row, 0)
        tot = tot + row

    nb = (tot + (BT - 1)) >> 7
    end_blk = _cumsum16(nb)
    off_slot = (end_blk - nb) * BT

    pre = off_slot + base
    for c in range(APW // 16):
        v = ev[pl.ds(c * 16, 16)]
        slotv[pl.ds(c * 16, 16)] = _take(pre, v) + rankv[pl.ds(c * 16, 16)]

    pltpu.sync_copy(slotv, slots_hbm.at[pl.ds(base_a, APW)])

    for c in range(8):
        t16 = w * TPW + ((c * 16 + _iota16()) >> 1)
        s16 = slotv[pl.ds(c * 16, 16)]
        s16 = jnp.minimum(jnp.maximum(s16, 0), S - 1)
        tok2d[c // 2, pl.ds((c % 2) * 16, 16)] = t16
        slot2d[c // 2, pl.ds((c % 2) * 16, 16)] = s16

    for rc in range(4):
        pltpu.async_copy(x_hbm.at[tok2d.at[rc]], rows, sem).wait()
        pltpu.async_copy(rows, xg_hbm.at[slot2d.at[rc]], sem).wait()

    @pl.when(w == 0)
    def _():
        nblocks = _bcast_at(end_blk, 7)
        for c in range(4):
            bvec = c * 16 + _iota16()
            acc = jnp.zeros((16,), jnp.int32)
            for e in range(E):
                acc = acc + jnp.where(bvec >= _bcast_at(end_blk, e), 1, 0)
            acc = jnp.minimum(acc, 7)
            val = jnp.where(bvec < NB, acc, jnp.where(bvec == NB, nblocks, 0))
            metav[pl.ds(c * 16, 16)] = val
        pltpu.sync_copy(metav, meta_hbm)


def _ffn_body(meta_ref, xg_ref, w1_ref, b1_ref, w2_ref, b2_ref, y_ref,
              acc_ref):
    h = pl.program_id(0)
    b = pl.program_id(1)
    nblocks = meta_ref[NB]

    @pl.when(b < nblocks)
    def _():
        part = jnp.dot(xg_ref[...], w1_ref[0]) + b1_ref[0]
        part = jnp.maximum(part, 0.0)
        part = jnp.dot(part, w2_ref[0])
        sl = pl.ds(b * BT, BT)

        @pl.when(h == 0)
        def _():
            acc_ref[sl, :] = part + b2_ref[0]

        @pl.when(h > 0)
        def _():
            acc_ref[sl, :] += part

    @pl.when(h == NH - 1)
    def _():
        y_ref[...] = acc_ref[pl.ds(b * BT, BT), :]


def _ffn(xg, w1, b1, w2, b2, meta):
    return pl.pallas_call(
        _ffn_body,
        grid_spec=pltpu.PrefetchScalarGridSpec(
            num_scalar_prefetch=1,
            grid=(NH, NB),
            in_specs=[
                pl.BlockSpec((BT, D), lambda h, b, m: (b, 0)),
                pl.BlockSpec((1, D, BH), lambda h, b, m: (m[b], 0, h)),
                pl.BlockSpec((1, 1, BH), lambda h, b, m: (m[b], 0, h)),
                pl.BlockSpec((1, BH, D), lambda h, b, m: (m[b], h, 0)),
                pl.BlockSpec((1, 1, D), lambda h, b, m: (m[b], 0, 0)),
            ],
            out_specs=pl.BlockSpec((BT, D), lambda h, b, m: (b, 0)),
            scratch_shapes=[pltpu.VMEM((S, D), jnp.float32)],
        ),
        out_shape=jax.ShapeDtypeStruct((S, D), jnp.float32),
        compiler_params=pltpu.CompilerParams(
            dimension_semantics=("arbitrary", "arbitrary"),
        ),
    )(meta, xg, w1, b1.reshape(E, 1, H), w2, b2.reshape(E, 1, D))


@functools.partial(
    pl.kernel, mesh=_sc_mesh,
    out_type=jax.ShapeDtypeStruct((N, D), jnp.float32),
    scratch_types=[
        pltpu.VMEM((APW,), jnp.int32),
        pltpu.VMEM((APW,), jnp.float32),
        pltpu.VMEM((32, D), jnp.float32),
        pltpu.VMEM((16, D), jnp.float32),
        pltpu.SemaphoreType.DMA,
    ],
)
def _sc_combine(y_hbm, slots_hbm, gates_hbm, out_hbm,
                slotv, gv, rows, outv, sem):
    w = lax.axis_index("s") * 2 + lax.axis_index("c")
    base_a = w * APW
    pltpu.sync_copy(slots_hbm.at[pl.ds(base_a, APW)], slotv)
    pltpu.sync_copy(gates_hbm.at[pl.ds(base_a, APW)], gv)

    for rc in range(4):
        pltpu.async_copy(y_hbm.at[slotv.at[pl.ds(rc * 32, 32)]], rows,
                         sem).wait()
        g0chunk = gv[pl.ds(rc * 32, 16)]
        g1chunk = gv[pl.ds(rc * 32 + 16, 16)]
        for i in range(16):
            gch = g0chunk if i < 8 else g1chunk
            j0 = (2 * i) % 16
            g0 = _take(gch, jnp.full((16,), j0, jnp.int32))
            g1 = _take(gch, jnp.full((16,), j0 + 1, jnp.int32))

            def body(j, _):
                sl = pl.ds(j * 16, 16)
                outv[i, sl] = rows[2 * i, sl] * g0 + rows[2 * i + 1, sl] * g1
                return 0

            lax.fori_loop(0, D // 16, body, 0)
        pltpu.sync_copy(outv, out_hbm.at[pl.ds(w * TPW + rc * 16, 16)])


def kernel(x, W_route, b_route, W_noise, b_noise, w1, b1, w2, b2, noise):
    indices, gates = _router(x, W_route, b_route, W_noise, b_noise, noise)
    idx_flat = indices.reshape(A)
    gates_flat = gates.reshape(A)
    cnt, rank = _sc_counts(idx_flat)
    xg, slots, meta = _sc_dispatch(idx_flat, rank, cnt, x)
    y = _ffn(xg, w1, b1, w2, b2, meta)
    return _sc_combine(y, slots, gates_flat)


# speedup vs baseline: 1.1269x; 1.1269x over previous
import functools
import jax
import jax.numpy as jnp
from jax import lax
from jax.experimental import pallas as pl
from jax.experimental.pallas import tpu as pltpu, tpu_sc as plsc

D = 1024
E = 8
H = 4096
N = 2048
A = 2 * N
BT = 128
NB = 40
S = NB * BT
BH = 512
NH = H // BH
NW = 32
APW = A // NW
TPW = N // NW

_sc_mesh = plsc.VectorSubcoreMesh(core_axis_name="c", subcore_axis_name="s")


def _iota16():
    return lax.iota(jnp.int32, 16)


def _take(arr, idx):
    return lax.gather(
        arr, idx[:, None],
        dimension_numbers=lax.GatherDimensionNumbers(
            offset_dims=(), collapsed_slice_dims=(0,), start_index_map=(0,)),
        slice_sizes=(1,),
        mode=lax.GatherScatterMode.PROMISE_IN_BOUNDS)


def _cumsum16(x):
    iota = _iota16()
    s = x
    for sh in (1, 2, 4, 8):
        idx = jnp.maximum(iota - sh, 0)
        s = s + jnp.where(iota >= sh, _take(s, idx), 0)
    return s


def _bcast_at(s, i):
    return _take(s, jnp.full((16,), i, jnp.int32))


def _router_body(x_ref, wr_ref, br_ref, wn_ref, bn_ref, noise_ref,
                 idx_ref, gates_ref):
    xb = x_ref[...]
    logits = jnp.dot(xb, wr_ref[...]) + br_ref[...]
    nlog = jnp.dot(xb, wn_ref[...]) + bn_ref[...]
    sp = jnp.maximum(nlog, 0.0) + jnp.log1p(jnp.exp(-jnp.abs(nlog)))
    nl = logits + noise_ref[...] * sp
    lane = jax.lax.broadcasted_iota(jnp.int32, nl.shape, 1)
    m1 = jnp.max(nl, axis=1, keepdims=True)
    i1 = jnp.min(jnp.where(nl == m1, lane, E), axis=1, keepdims=True)
    nl2 = jnp.where(lane == i1, -jnp.inf, nl)
    m2 = jnp.max(nl2, axis=1, keepdims=True)
    i2 = jnp.min(jnp.where(nl2 == m2, lane, E), axis=1, keepdims=True)
    e2 = jnp.exp(m2 - m1)
    denom = 1.0 + e2
    idx_ref[...] = jnp.concatenate([i1, i2], axis=1)
    gates_ref[...] = jnp.concatenate([1.0 / denom, e2 / denom], axis=1)


def _router(x, W_route, b_route, W_noise, b_noise, noise):
    return pl.pallas_call(
        _router_body,
        out_shape=(jax.ShapeDtypeStruct((N, 2), jnp.int32),
                   jax.ShapeDtypeStruct((N, 2), jnp.float32)),
    )(x, W_route, b_route.reshape(1, E), W_noise, b_noise.reshape(1, E),
      noise)


@functools.partial(
    pl.kernel, mesh=_sc_mesh,
    out_type=(jax.ShapeDtypeStruct((NW, 16), jnp.int32),
              jax.ShapeDtypeStruct((A,), jnp.int32)),
    scratch_types=[
        pltpu.VMEM((APW,), jnp.int32),
        pltpu.VMEM((APW,), jnp.int32),
        pltpu.VMEM((16,), jnp.int32),
        pltpu.SemaphoreType.DMA,
    ],
)
def _sc_counts(idx_hbm, cnt_hbm, rank_hbm, ev, rankv, cntv, sem):
    w = lax.axis_index("s") * 2 + lax.axis_index("c")
    base_a = w * APW
    lanes = _iota16()
    pltpu.sync_copy(idx_hbm.at[pl.ds(base_a, APW)], ev)

    run = jnp.zeros((16,), jnp.int32)
    for c in range(APW // 16):
        v = ev[pl.ds(c * 16, 16)]
        base_r = _take(run, v)
        rank = jnp.zeros((16,), jnp.int32)
        for e in range(E):
            m = v == e
            cs = _cumsum16(jnp.where(m, 1, 0))
            rank = rank + jnp.where(m, cs - 1, 0)
            run = run + jnp.where(lanes == e, _bcast_at(cs, 15), 0)
        rankv[pl.ds(c * 16, 16)] = base_r + rank

    cntv[...] = run
    pltpu.sync_copy(cntv, cnt_hbm.at[w])
    pltpu.sync_copy(rankv, rank_hbm.at[pl.ds(base_a, APW)])


@functools.partial(
    pl.kernel, mesh=_sc_mesh,
    out_type=(jax.ShapeDtypeStruct((S, D), jnp.float32),
              jax.ShapeDtypeStruct((A,), jnp.int32),
              jax.ShapeDtypeStruct((64,), jnp.int32)),
    scratch_types=[
        pltpu.VMEM((APW,), jnp.int32),
        pltpu.VMEM((APW,), jnp.int32),
        pltpu.VMEM((NW, 16), jnp.int32),
        pltpu.VMEM((APW,), jnp.int32),
        pltpu.VMEM((4, 32), jnp.int32),
        pltpu.VMEM((4, 32), jnp.int32),
        pltpu.VMEM((32, D), jnp.float32),
        pltpu.VMEM((64,), jnp.int32),
        pltpu.SemaphoreType.DMA,
    ],
)
def _sc_dispatch(idx_hbm, rank_hbm, cnt_hbm, x_hbm, xg_hbm, slots_hbm,
                 meta_hbm, ev, rankv, allcnt, slotv, tok2d, slot2d, rows,
                 metav, sem):
    w = lax.axis_index("s") * 2 + lax.axis_index("c")
    base_a = w * APW
    pltpu.sync_copy(idx_hbm.at[pl.ds(base_a, APW)], ev)
    pltpu.sync_copy(rank_hbm.at[pl.ds(base_a, APW)], rankv)
    pltpu.sync_copy(cnt_hbm, allcnt)

    base = jnp.zeros((16,), jnp.int32)
    tot = jnp.zeros((16,), jnp.int32)
    for wp in range(NW):
        row = allcnt[wp]
        base = base + jnp.where(wp < w, row, 0)
        tot = tot + row

    nb = (tot + (BT - 1)) >> 7
    end_blk = _cumsum16(nb)
    off_slot = (end_blk - nb) * BT

    pre = off_slot + base
    for c in range(APW // 16):
        v = ev[pl.ds(c * 16, 16)]
        slotv[pl.ds(c * 16, 16)] = _take(pre, v) + rankv[pl.ds(c * 16, 16)]

    pltpu.sync_copy(slotv, slots_hbm.at[pl.ds(base_a, APW)])

    for c in range(8):
        t16 = w * TPW + ((c * 16 + _iota16()) >> 1)
        s16 = slotv[pl.ds(c * 16, 16)]
        s16 = jnp.minimum(jnp.maximum(s16, 0), S - 1)
        tok2d[c // 2, pl.ds((c % 2) * 16, 16)] = t16
        slot2d[c // 2, pl.ds((c % 2) * 16, 16)] = s16

    for rc in range(4):
        pltpu.async_copy(x_hbm.at[tok2d.at[rc]], rows, sem).wait()
        pltpu.async_copy(rows, xg_hbm.at[slot2d.at[rc]], sem).wait()

    @pl.when(w == 0)
    def _():
        nblocks = _bcast_at(end_blk, 7)
        for c in range(4):
            bvec = c * 16 + _iota16()
            acc = jnp.zeros((16,), jnp.int32)
            for e in range(E):
                acc = acc + jnp.where(bvec >= _bcast_at(end_blk, e), 1, 0)
            acc = jnp.minimum(acc, 7)
            val = jnp.where(bvec < NB, acc, jnp.where(bvec == NB, nblocks, 0))
            metav[pl.ds(c * 16, 16)] = val
        pltpu.sync_copy(metav, meta_hbm)


def _ffn_body(meta_ref, xg_ref, w1_ref, b1_ref, w2_ref, b2_ref, y_ref,
              acc_ref, sem):
    h = pl.program_id(0)
    b = pl.program_id(1)
    nblocks = meta_ref[NB]

    @pl.when(b < nblocks)
    def _():
        sl = pl.ds(b * BT, BT)
        part = jnp.dot(xg_ref[sl, :], w1_ref[0]) + b1_ref[0]
        part = jnp.maximum(part, 0.0)
        part = jnp.dot(part, w2_ref[0])

        @pl.when(h == 0)
        def _():
            acc_ref[sl, :] = part + b2_ref[0]

        @pl.when(h > 0)
        def _():
            acc_ref[sl, :] += part

        @pl.when(h == NH - 1)
        def _():
            cp = pltpu.make_async_copy(acc_ref.at[sl, :], y_ref.at[sl, :], sem)
            cp.start()
            cp.wait()


def _ffn(xg, w1, b1, w2, b2, meta):
    return pl.pallas_call(
        _ffn_body,
        grid_spec=pltpu.PrefetchScalarGridSpec(
            num_scalar_prefetch=1,
            grid=(NH, NB),
            in_specs=[
                pl.BlockSpec((S, D), lambda h, b, m: (0, 0)),
                pl.BlockSpec((1, D, BH), lambda h, b, m: (m[b], 0, h)),
                pl.BlockSpec((1, 1, BH), lambda h, b, m: (m[b], 0, h)),
                pl.BlockSpec((1, BH, D), lambda h, b, m: (m[b], h, 0)),
                pl.BlockSpec((1, 1, D), lambda h, b, m: (m[b], 0, 0)),
            ],
            out_specs=pl.BlockSpec(memory_space=pl.ANY),
            scratch_shapes=[pltpu.VMEM((S, D), jnp.float32),
                            pltpu.SemaphoreType.DMA],
        ),
        out_shape=jax.ShapeDtypeStruct((S, D), jnp.float32),
        compiler_params=pltpu.CompilerParams(
            dimension_semantics=("arbitrary", "arbitrary"),
        ),
    )(meta, xg, w1, b1.reshape(E, 1, H), w2, b2.reshape(E, 1, D))


@functools.partial(
    pl.kernel, mesh=_sc_mesh,
    out_type=jax.ShapeDtypeStruct((N, D), jnp.float32),
    scratch_types=[
        pltpu.VMEM((APW,), jnp.int32),
        pltpu.VMEM((APW,), jnp.float32),
        pltpu.VMEM((32, D), jnp.float32),
        pltpu.VMEM((16, D), jnp.float32),
        pltpu.SemaphoreType.DMA,
    ],
)
def _sc_combine(y_hbm, slots_hbm, gates_hbm, out_hbm,
                slotv, gv, rows, outv, sem):
    w = lax.axis_index("s") * 2 + lax.axis_index("c")
    base_a = w * APW
    pltpu.sync_copy(slots_hbm.at[pl.ds(base_a, APW)], slotv)
    pltpu.sync_copy(gates_hbm.at[pl.ds(base_a, APW)], gv)

    for rc in range(4):
        pltpu.async_copy(y_hbm.at[slotv.at[pl.ds(rc * 32, 32)]], rows,
                         sem).wait()
        g0chunk = gv[pl.ds(rc * 32, 16)]
        g1chunk = gv[pl.ds(rc * 32 + 16, 16)]
        for i in range(16):
            gch = g0chunk if i < 8 else g1chunk
            j0 = (2 * i) % 16
            g0 = _take(gch, jnp.full((16,), j0, jnp.int32))
            g1 = _take(gch, jnp.full((16,), j0 + 1, jnp.int32))

            def body(j, _):
                sl = pl.ds(j * 16, 16)
                outv[i, sl] = rows[2 * i, sl] * g0 + rows[2 * i + 1, sl] * g1
                return 0

            lax.fori_loop(0, D // 16, body, 0)
        pltpu.sync_copy(outv, out_hbm.at[pl.ds(w * TPW + rc * 16, 16)])


def kernel(x, W_route, b_route, W_noise, b_noise, w1, b1, w2, b2, noise):
    indices, gates = _router(x, W_route, b_route, W_noise, b_noise, noise)
    idx_flat = indices.reshape(A)
    gates_flat = gates.reshape(A)
    cnt, rank = _sc_counts(idx_flat)
    xg, slots, meta = _sc_dispatch(idx_flat, rank, cnt, x)
    y = _ffn(xg, w1, b1, w2, b2, meta)
    return _sc_combine(y, slots, gates_flat)


# speedup vs baseline: 1.4256x; 1.2651x over previous
import functools
import jax
import jax.numpy as jnp
from jax import lax
from jax.experimental import pallas as pl
from jax.experimental.pallas import tpu as pltpu, tpu_sc as plsc

D = 1024
E = 8
H = 4096
N = 2048
A = 2 * N
BT = 128
NB = 40
S = NB * BT
BH = 1024
NH = H // BH
NW = 32
APW = A // NW
TPW = N // NW

_sc_mesh = plsc.VectorSubcoreMesh(core_axis_name="c", subcore_axis_name="s")


def _iota16():
    return lax.iota(jnp.int32, 16)


def _take(arr, idx):
    return lax.gather(
        arr, idx[:, None],
        dimension_numbers=lax.GatherDimensionNumbers(
            offset_dims=(), collapsed_slice_dims=(0,), start_index_map=(0,)),
        slice_sizes=(1,),
        mode=lax.GatherScatterMode.PROMISE_IN_BOUNDS)


def _cumsum16(x):
    iota = _iota16()
    s = x
    for sh in (1, 2, 4, 8):
        idx = jnp.maximum(iota - sh, 0)
        s = s + jnp.where(iota >= sh, _take(s, idx), 0)
    return s


def _bcast_at(s, i):
    return _take(s, jnp.full((16,), i, jnp.int32))


def _router_body(x_ref, wr_ref, br_ref, wn_ref, bn_ref, noise_ref,
                 idx_ref, gates_ref):
    xb = x_ref[...]
    logits = jnp.dot(xb, wr_ref[...]) + br_ref[...]
    nlog = jnp.dot(xb, wn_ref[...]) + bn_ref[...]
    sp = jnp.maximum(nlog, 0.0) + jnp.log1p(jnp.exp(-jnp.abs(nlog)))
    nl = logits + noise_ref[...] * sp
    lane = jax.lax.broadcasted_iota(jnp.int32, nl.shape, 1)
    m1 = jnp.max(nl, axis=1, keepdims=True)
    i1 = jnp.min(jnp.where(nl == m1, lane, E), axis=1, keepdims=True)
    nl2 = jnp.where(lane == i1, -jnp.inf, nl)
    m2 = jnp.max(nl2, axis=1, keepdims=True)
    i2 = jnp.min(jnp.where(nl2 == m2, lane, E), axis=1, keepdims=True)
    e2 = jnp.exp(m2 - m1)
    denom = 1.0 + e2
    idx_ref[...] = jnp.concatenate([i1, i2], axis=1)
    gates_ref[...] = jnp.concatenate([1.0 / denom, e2 / denom], axis=1)


def _router(x, W_route, b_route, W_noise, b_noise, noise):
    return pl.pallas_call(
        _router_body,
        out_shape=(jax.ShapeDtypeStruct((N, 2), jnp.int32),
                   jax.ShapeDtypeStruct((N, 2), jnp.float32)),
    )(x, W_route, b_route.reshape(1, E), W_noise, b_noise.reshape(1, E),
      noise)


@functools.partial(
    pl.kernel, mesh=_sc_mesh,
    out_type=(jax.ShapeDtypeStruct((NW, 16), jnp.int32),
              jax.ShapeDtypeStruct((A,), jnp.int32)),
    scratch_types=[
        pltpu.VMEM((APW,), jnp.int32),
        pltpu.VMEM((APW,), jnp.int32),
        pltpu.VMEM((16,), jnp.int32),
        pltpu.SemaphoreType.DMA,
    ],
)
def _sc_counts(idx_hbm, cnt_hbm, rank_hbm, ev, rankv, cntv, sem):
    w = lax.axis_index("s") * 2 + lax.axis_index("c")
    base_a = w * APW
    lanes = _iota16()
    pltpu.sync_copy(idx_hbm.at[pl.ds(base_a, APW)], ev)

    run = jnp.zeros((16,), jnp.int32)
    for c in range(APW // 16):
        v = ev[pl.ds(c * 16, 16)]
        base_r = _take(run, v)
        rank = jnp.zeros((16,), jnp.int32)
        for e in range(E):
            m = v == e
            cs = _cumsum16(jnp.where(m, 1, 0))
            rank = rank + jnp.where(m, cs - 1, 0)
            run = run + jnp.where(lanes == e, _bcast_at(cs, 15), 0)
        rankv[pl.ds(c * 16, 16)] = base_r + rank

    cntv[...] = run
    pltpu.sync_copy(cntv, cnt_hbm.at[w])
    pltpu.sync_copy(rankv, rank_hbm.at[pl.ds(base_a, APW)])


@functools.partial(
    pl.kernel, mesh=_sc_mesh,
    out_type=(jax.ShapeDtypeStruct((S, D), jnp.float32),
              jax.ShapeDtypeStruct((A,), jnp.int32),
              jax.ShapeDtypeStruct((64,), jnp.int32)),
    scratch_types=[
        pltpu.VMEM((APW,), jnp.int32),
        pltpu.VMEM((APW,), jnp.int32),
        pltpu.VMEM((NW, 16), jnp.int32),
        pltpu.VMEM((APW,), jnp.int32),
        pltpu.VMEM((4, 32), jnp.int32),
        pltpu.VMEM((4, 32), jnp.int32),
        pltpu.VMEM((32, D), jnp.float32),
        pltpu.VMEM((64,), jnp.int32),
        pltpu.SemaphoreType.DMA,
    ],
)
def _sc_dispatch(idx_hbm, rank_hbm, cnt_hbm, x_hbm, xg_hbm, slots_hbm,
                 meta_hbm, ev, rankv, allcnt, slotv, tok2d, slot2d, rows,
                 metav, sem):
    w = lax.axis_index("s") * 2 + lax.axis_index("c")
    base_a = w * APW
    pltpu.sync_copy(idx_hbm.at[pl.ds(base_a, APW)], ev)
    pltpu.sync_copy(rank_hbm.at[pl.ds(base_a, APW)], rankv)
    pltpu.sync_copy(cnt_hbm, allcnt)

    base = jnp.zeros((16,), jnp.int32)
    tot = jnp.zeros((16,), jnp.int32)
    for wp in range(NW):
        row = allcnt[wp]
        base = base + jnp.where(wp < w, row, 0)
        tot = tot + row

    nb = (tot + (BT - 1)) >> 7
    end_blk = _cumsum16(nb)
    off_slot = (end_blk - nb) * BT

    pre = off_slot + base
    for c in range(APW // 16):
        v = ev[pl.ds(c * 16, 16)]
        slotv[pl.ds(c * 16, 16)] = _take(pre, v) + rankv[pl.ds(c * 16, 16)]

    pltpu.sync_copy(slotv, slots_hbm.at[pl.ds(base_a, APW)])

    for c in range(8):
        t16 = w * TPW + ((c * 16 + _iota16()) >> 1)
        s16 = slotv[pl.ds(c * 16, 16)]
        s16 = jnp.minimum(jnp.maximum(s16, 0), S - 1)
        tok2d[c // 2, pl.ds((c % 2) * 16, 16)] = t16
        slot2d[c // 2, pl.ds((c % 2) * 16, 16)] = s16

    for rc in range(4):
        pltpu.async_copy(x_hbm.at[tok2d.at[rc]], rows, sem).wait()
        pltpu.async_copy(rows, xg_hbm.at[slot2d.at[rc]], sem).wait()

    @pl.when(w == 0)
    def _():
        nblocks = _bcast_at(end_blk, 7)
        for c in range(4):
            bvec = c * 16 + _iota16()
            acc = jnp.zeros((16,), jnp.int32)
            for e in range(E):
                acc = acc + jnp.where(bvec >= _bcast_at(end_blk, e), 1, 0)
            acc = jnp.minimum(acc, 7)
            val = jnp.where(bvec < NB, acc, jnp.where(bvec == NB, nblocks, 0))
            metav[pl.ds(c * 16, 16)] = val
        pltpu.sync_copy(metav, meta_hbm)


def _ffn_body(meta_ref, xg_ref, w1_ref, b1_ref, w2_ref, b2_ref, y_ref,
              acc_ref, sem):
    h = pl.program_id(0)
    b = pl.program_id(1)
    nblocks = meta_ref[NB]

    @pl.when(b < nblocks)
    def _():
        sl = pl.ds(b * BT, BT)
        part = jnp.dot(xg_ref[sl, :], w1_ref[0].astype(jnp.bfloat16),
                       preferred_element_type=jnp.float32) + b1_ref[0]
        part = jnp.maximum(part, 0.0)
        part = jnp.dot(part, w2_ref[0])

        @pl.when(h == 0)
        def _():
            acc_ref[sl, :] = part + b2_ref[0]

        @pl.when(h > 0)
        def _():
            acc_ref[sl, :] += part

        @pl.when(h == NH - 1)
        def _():
            cp = pltpu.make_async_copy(acc_ref.at[sl, :], y_ref.at[sl, :], sem)
            cp.start()
            cp.wait()


def _ffn(xg, w1, b1, w2, b2, meta):
    return pl.pallas_call(
        _ffn_body,
        grid_spec=pltpu.PrefetchScalarGridSpec(
            num_scalar_prefetch=1,
            grid=(NH, NB),
            in_specs=[
                pl.BlockSpec((S, D), lambda h, b, m: (0, 0)),
                pl.BlockSpec((1, D, BH), lambda h, b, m: (m[b], 0, h)),
                pl.BlockSpec((1, 1, BH), lambda h, b, m: (m[b], 0, h)),
                pl.BlockSpec((1, BH, D), lambda h, b, m: (m[b], h, 0)),
                pl.BlockSpec((1, 1, D), lambda h, b, m: (m[b], 0, 0)),
            ],
            out_specs=pl.BlockSpec(memory_space=pl.ANY),
            scratch_shapes=[pltpu.VMEM((S, D), jnp.float32),
                            pltpu.SemaphoreType.DMA],
        ),
        out_shape=jax.ShapeDtypeStruct((S, D), jnp.float32),
        compiler_params=pltpu.CompilerParams(
            dimension_semantics=("arbitrary", "arbitrary"),
        ),
    )(meta, xg.astype(jnp.bfloat16), w1, b1.reshape(E, 1, H), w2,
      b2.reshape(E, 1, D))


@functools.partial(
    pl.kernel, mesh=_sc_mesh,
    out_type=jax.ShapeDtypeStruct((N, D), jnp.float32),
    scratch_types=[
        pltpu.VMEM((APW,), jnp.int32),
        pltpu.VMEM((APW,), jnp.float32),
        pltpu.VMEM((32, D), jnp.float32),
        pltpu.VMEM((16, D), jnp.float32),
        pltpu.SemaphoreType.DMA,
    ],
)
def _sc_combine(y_hbm, slots_hbm, gates_hbm, out_hbm,
                slotv, gv, rows, outv, sem):
    w = lax.axis_index("s") * 2 + lax.axis_index("c")
    base_a = w * APW
    pltpu.sync_copy(slots_hbm.at[pl.ds(base_a, APW)], slotv)
    pltpu.sync_copy(gates_hbm.at[pl.ds(base_a, APW)], gv)

    for rc in range(4):
        pltpu.async_copy(y_hbm.at[slotv.at[pl.ds(rc * 32, 32)]], rows,
                         sem).wait()
        g0chunk = gv[pl.ds(rc * 32, 16)]
        g1chunk = gv[pl.ds(rc * 32 + 16, 16)]
        for i in range(16):
            gch = g0chunk if i < 8 else g1chunk
            j0 = (2 * i) % 16
            g0 = _take(gch, jnp.full((16,), j0, jnp.int32))
            g1 = _take(gch, jnp.full((16,), j0 + 1, jnp.int32))

            def body(j, _):
                sl = pl.ds(j * 16, 16)
                outv[i, sl] = rows[2 * i, sl] * g0 + rows[2 * i + 1, sl] * g1
                return 0

            lax.fori_loop(0, D // 16, body, 0)
        pltpu.sync_copy(outv, out_hbm.at[pl.ds(w * TPW + rc * 16, 16)])


def kernel(x, W_route, b_route, W_noise, b_noise, w1, b1, w2, b2, noise):
    indices, gates = _router(x, W_route, b_route, W_noise, b_noise, noise)
    idx_flat = indices.reshape(A)
    gates_flat = gates.reshape(A)
    cnt, rank = _sc_counts(idx_flat)
    xg, slots, meta = _sc_dispatch(idx_flat, rank, cnt, x)
    y = _ffn(xg, w1, b1, w2, b2, meta)
    return _sc_combine(y, slots, gates_flat)


# speedup vs baseline: 1.6135x; 1.1318x over previous
import functools
import jax
import jax.numpy as jnp
from jax import lax
from jax.experimental import pallas as pl
from jax.experimental.pallas import tpu as pltpu, tpu_sc as plsc

D = 1024
E = 8
H = 4096
N = 2048
A = 2 * N
BT = 256
NB = 24
S = NB * BT
BH = 1024
NH = H // BH
NW = 32
APW = A // NW
TPW = N // NW

_sc_mesh = plsc.VectorSubcoreMesh(core_axis_name="c", subcore_axis_name="s")


def _iota16():
    return lax.iota(jnp.int32, 16)


def _take(arr, idx):
    return lax.gather(
        arr, idx[:, None],
        dimension_numbers=lax.GatherDimensionNumbers(
            offset_dims=(), collapsed_slice_dims=(0,), start_index_map=(0,)),
        slice_sizes=(1,),
        mode=lax.GatherScatterMode.PROMISE_IN_BOUNDS)


def _cumsum16(x):
    iota = _iota16()
    s = x
    for sh in (1, 2, 4, 8):
        idx = jnp.maximum(iota - sh, 0)
        s = s + jnp.where(iota >= sh, _take(s, idx), 0)
    return s


def _bcast_at(s, i):
    return _take(s, jnp.full((16,), i, jnp.int32))


def _router_body(x_ref, wr_ref, br_ref, wn_ref, bn_ref, noise_ref,
                 idx_ref, gates_ref):
    xb = x_ref[...]
    logits = jnp.dot(xb, wr_ref[...]) + br_ref[...]
    nlog = jnp.dot(xb, wn_ref[...]) + bn_ref[...]
    sp = jnp.maximum(nlog, 0.0) + jnp.log1p(jnp.exp(-jnp.abs(nlog)))
    nl = logits + noise_ref[...] * sp
    lane = jax.lax.broadcasted_iota(jnp.int32, nl.shape, 1)
    m1 = jnp.max(nl, axis=1, keepdims=True)
    i1 = jnp.min(jnp.where(nl == m1, lane, E), axis=1, keepdims=True)
    nl2 = jnp.where(lane == i1, -jnp.inf, nl)
    m2 = jnp.max(nl2, axis=1, keepdims=True)
    i2 = jnp.min(jnp.where(nl2 == m2, lane, E), axis=1, keepdims=True)
    e2 = jnp.exp(m2 - m1)
    denom = 1.0 + e2
    idx_ref[...] = jnp.concatenate([i1, i2], axis=1)
    gates_ref[...] = jnp.concatenate([1.0 / denom, e2 / denom], axis=1)


def _router(x, W_route, b_route, W_noise, b_noise, noise):
    return pl.pallas_call(
        _router_body,
        out_shape=(jax.ShapeDtypeStruct((N, 2), jnp.int32),
                   jax.ShapeDtypeStruct((N, 2), jnp.float32)),
    )(x, W_route, b_route.reshape(1, E), W_noise, b_noise.reshape(1, E),
      noise)


@functools.partial(
    pl.kernel, mesh=_sc_mesh,
    out_type=(jax.ShapeDtypeStruct((NW, 16), jnp.int32),
              jax.ShapeDtypeStruct((A,), jnp.int32)),
    scratch_types=[
        pltpu.VMEM((APW,), jnp.int32),
        pltpu.VMEM((APW,), jnp.int32),
        pltpu.VMEM((16,), jnp.int32),
        pltpu.SemaphoreType.DMA,
    ],
)
def _sc_counts(idx_hbm, cnt_hbm, rank_hbm, ev, rankv, cntv, sem):
    w = lax.axis_index("s") * 2 + lax.axis_index("c")
    base_a = w * APW
    lanes = _iota16()
    pltpu.sync_copy(idx_hbm.at[pl.ds(base_a, APW)], ev)

    run = jnp.zeros((16,), jnp.int32)
    for c in range(APW // 16):
        v = ev[pl.ds(c * 16, 16)]
        base_r = _take(run, v)
        rank = jnp.zeros((16,), jnp.int32)
        for e in range(E):
            m = v == e
            cs = _cumsum16(jnp.where(m, 1, 0))
            rank = rank + jnp.where(m, cs - 1, 0)
            run = run + jnp.where(lanes == e, _bcast_at(cs, 15), 0)
        rankv[pl.ds(c * 16, 16)] = base_r + rank

    cntv[...] = run
    pltpu.sync_copy(cntv, cnt_hbm.at[w])
    pltpu.sync_copy(rankv, rank_hbm.at[pl.ds(base_a, APW)])


@functools.partial(
    pl.kernel, mesh=_sc_mesh,
    out_type=(jax.ShapeDtypeStruct((S, D), jnp.float32),
              jax.ShapeDtypeStruct((A,), jnp.int32),
              jax.ShapeDtypeStruct((64,), jnp.int32)),
    scratch_types=[
        pltpu.VMEM((APW,), jnp.int32),
        pltpu.VMEM((APW,), jnp.int32),
        pltpu.VMEM((NW, 16), jnp.int32),
        pltpu.VMEM((APW,), jnp.int32),
        pltpu.VMEM((4, 32), jnp.int32),
        pltpu.VMEM((4, 32), jnp.int32),
        pltpu.VMEM((32, D), jnp.float32),
        pltpu.VMEM((64,), jnp.int32),
        pltpu.SemaphoreType.DMA,
    ],
)
def _sc_dispatch(idx_hbm, rank_hbm, cnt_hbm, x_hbm, xg_hbm, slots_hbm,
                 meta_hbm, ev, rankv, allcnt, slotv, tok2d, slot2d, rows,
                 metav, sem):
    w = lax.axis_index("s") * 2 + lax.axis_index("c")
    base_a = w * APW
    pltpu.sync_copy(idx_hbm.at[pl.ds(base_a, APW)], ev)
    pltpu.sync_copy(rank_hbm.at[pl.ds(base_a, APW)], rankv)
    pltpu.sync_copy(cnt_hbm, allcnt)

    base = jnp.zeros((16,), jnp.int32)
    tot = jnp.zeros((16,), jnp.int32)
    for wp in range(NW):
        row = allcnt[wp]
        base = base + jnp.where(wp < w, row, 0)
        tot = tot + row

    nb = (tot + (BT - 1)) >> 8
    end_blk = _cumsum16(nb)
    off_slot = (end_blk - nb) * BT

    pre = off_slot + base
    for c in range(APW // 16):
        v = ev[pl.ds(c * 16, 16)]
        slotv[pl.ds(c * 16, 16)] = _take(pre, v) + rankv[pl.ds(c * 16, 16)]

    pltpu.sync_copy(slotv, slots_hbm.at[pl.ds(base_a, APW)])

    for c in range(8):
        t16 = w * TPW + ((c * 16 + _iota16()) >> 1)
        s16 = slotv[pl.ds(c * 16, 16)]
        s16 = jnp.minimum(jnp.maximum(s16, 0), S - 1)
        tok2d[c // 2, pl.ds((c % 2) * 16, 16)] = t16
        slot2d[c // 2, pl.ds((c % 2) * 16, 16)] = s16

    for rc in range(4):
        pltpu.async_copy(x_hbm.at[tok2d.at[rc]], rows, sem).wait()
        pltpu.async_copy(rows, xg_hbm.at[slot2d.at[rc]], sem).wait()

    @pl.when(w == 0)
    def _():
        nblocks = _bcast_at(end_blk, 7)
        for c in range(4):
            bvec = c * 16 + _iota16()
            acc = jnp.zeros((16,), jnp.int32)
            for e in range(E):
                acc = acc + jnp.where(bvec >= _bcast_at(end_blk, e), 1, 0)
            acc = jnp.minimum(acc, 7)
            val = jnp.where(bvec < NB, acc, jnp.where(bvec == NB, nblocks, 0))
            metav[pl.ds(c * 16, 16)] = val
        pltpu.sync_copy(metav, meta_hbm)


def _ffn_body(meta_ref, xg_ref, w1_ref, b1_ref, w2_ref, b2_ref, y_ref,
              acc_ref, sem):
    h = pl.program_id(0)
    b = pl.program_id(1)
    nblocks = meta_ref[NB]

    @pl.when(b < nblocks)
    def _():
        sl = pl.ds(b * BT, BT)
        part = jnp.dot(xg_ref[sl, :], w1_ref[0].astype(jnp.bfloat16),
                       preferred_element_type=jnp.float32) + b1_ref[0]
        part = jnp.maximum(part, 0.0)
        part = jnp.dot(part, w2_ref[0])

        @pl.when(h == 0)
        def _():
            acc_ref[sl, :] = part + b2_ref[0]

        @pl.when(h > 0)
        def _():
            acc_ref[sl, :] += part

        @pl.when(h == NH - 1)
        def _():
            cp = pltpu.make_async_copy(acc_ref.at[sl, :], y_ref.at[sl, :], sem)
            cp.start()
            cp.wait()


def _ffn(xg, w1, b1, w2, b2, meta):
    return pl.pallas_call(
        _ffn_body,
        grid_spec=pltpu.PrefetchScalarGridSpec(
            num_scalar_prefetch=1,
            grid=(NH, NB),
            in_specs=[
                pl.BlockSpec((S, D), lambda h, b, m: (0, 0)),
                pl.BlockSpec((1, D, BH), lambda h, b, m: (m[b], 0, h)),
                pl.BlockSpec((1, 1, BH), lambda h, b, m: (m[b], 0, h)),
                pl.BlockSpec((1, BH, D), lambda h, b, m: (m[b], h, 0)),
                pl.BlockSpec((1, 1, D), lambda h, b, m: (m[b], 0, 0)),
            ],
            out_specs=pl.BlockSpec(memory_space=pl.ANY),
            scratch_shapes=[pltpu.VMEM((S, D), jnp.float32),
                            pltpu.SemaphoreType.DMA],
        ),
        out_shape=jax.ShapeDtypeStruct((S, D), jnp.float32),
        compiler_params=pltpu.CompilerParams(
            dimension_semantics=("arbitrary", "arbitrary"),
        ),
    )(meta, xg.astype(jnp.bfloat16), w1, b1.reshape(E, 1, H), w2,
      b2.reshape(E, 1, D))


@functools.partial(
    pl.kernel, mesh=_sc_mesh,
    out_type=jax.ShapeDtypeStruct((N, D), jnp.float32),
    scratch_types=[
        pltpu.VMEM((APW,), jnp.int32),
        pltpu.VMEM((APW,), jnp.float32),
        pltpu.VMEM((32, D), jnp.float32),
        pltpu.VMEM((16, D), jnp.float32),
        pltpu.SemaphoreType.DMA,
    ],
)
def _sc_combine(y_hbm, slots_hbm, gates_hbm, out_hbm,
                slotv, gv, rows, outv, sem):
    w = lax.axis_index("s") * 2 + lax.axis_index("c")
    base_a = w * APW
    pltpu.sync_copy(slots_hbm.at[pl.ds(base_a, APW)], slotv)
    pltpu.sync_copy(gates_hbm.at[pl.ds(base_a, APW)], gv)

    for rc in range(4):
        pltpu.async_copy(y_hbm.at[slotv.at[pl.ds(rc * 32, 32)]], rows,
                         sem).wait()
        g0chunk = gv[pl.ds(rc * 32, 16)]
        g1chunk = gv[pl.ds(rc * 32 + 16, 16)]
        for i in range(16):
            gch = g0chunk if i < 8 else g1chunk
            j0 = (2 * i) % 16
            g0 = _take(gch, jnp.full((16,), j0, jnp.int32))
            g1 = _take(gch, jnp.full((16,), j0 + 1, jnp.int32))

            def body(j, _):
                sl = pl.ds(j * 16, 16)
                outv[i, sl] = rows[2 * i, sl] * g0 + rows[2 * i + 1, sl] * g1
                return 0

            lax.fori_loop(0, D // 16, body, 0)
        pltpu.sync_copy(outv, out_hbm.at[pl.ds(w * TPW + rc * 16, 16)])


def kernel(x, W_route, b_route, W_noise, b_noise, w1, b1, w2, b2, noise):
    indices, gates = _router(x, W_route, b_route, W_noise, b_noise, noise)
    idx_flat = indices.reshape(A)
    gates_flat = gates.reshape(A)
    cnt, rank = _sc_counts(idx_flat)
    xg, slots, meta = _sc_dispatch(idx_flat, rank, cnt, x)
    y = _ffn(xg, w1, b1, w2, b2, meta)
    return _sc_combine(y, slots, gates_flat)
